# baseline (speedup 1.0000x reference)
import functools

import jax
import jax.numpy as jnp
from jax import lax
from jax.experimental import pallas as pl
from jax.experimental.pallas import tpu as pltpu
from jax.experimental.pallas import tpu_sc as plsc

J, M, S, D = 1024, 128, 128, 128
_EPS = 1e-8

_NC, _NS = 2, 16
_NW = _NC * _NS
_NH = _NW // 2
_SM_PER_W = M // _NH
_JOB_PER_W = J // _NW
_L = 16
_CHUNK_JOBS = 2
_NCH = _JOB_PER_W // _CHUNK_JOBS


def _wid():
    return lax.axis_index("s") * _NC + lax.axis_index("c")


def _gather_body(jidx_hbm, midx_hbm, sidx_hbm, jtab_hbm, mtab_hbm, stab_hbm,
                 jout_hbm, mout_hbm, sout_hbm,
                 jidx_v, jrows_v, midx_v, mrows_v, sidx_v, srows_v,
                 sem_a, sem_b, sem_c):
    wid = _wid()
    hid = lax.rem(wid, _NH)
    jb = pl.multiple_of(wid * _JOB_PER_W, 8)
    sb = pl.multiple_of(hid * _SM_PER_W, 8)

    c1 = pltpu.async_copy(jidx_hbm.at[pl.ds(jb, _JOB_PER_W)], jidx_v, sem_a)
    c2 = pltpu.async_copy(midx_hbm.at[hid], midx_v, sem_b)
    c3 = pltpu.async_copy(sidx_hbm.at[hid], sidx_v, sem_c)
    c1.wait()
    c2.wait()
    c3.wait()
    g1 = pltpu.async_copy(jtab_hbm.at[jidx_v], jrows_v, sem_a)
    g2 = pltpu.async_copy(mtab_hbm.at[midx_v], mrows_v, sem_b)
    g3 = pltpu.async_copy(stab_hbm.at[sidx_v], srows_v, sem_c)
    g1.wait()
    g2.wait()
    g3.wait()
    w1 = pltpu.async_copy(jrows_v, jout_hbm.at[pl.ds(jb, _JOB_PER_W)], sem_a)
    w2 = pltpu.async_copy(mrows_v, mout_hbm.at[pl.ds(sb, _SM_PER_W)], sem_b)
    w3 = pltpu.async_copy(srows_v, sout_hbm.at[pl.ds(sb, _SM_PER_W)], sem_c)
    w1.wait()
    w2.wait()
    w3.wait()


@functools.cache
def _gather_sc():
    return pl.kernel(
        _gather_body,
        mesh=plsc.VectorSubcoreMesh(core_axis_name="c", subcore_axis_name="s"),
        out_type=[
            jax.ShapeDtypeStruct((J, D), jnp.float32),
            jax.ShapeDtypeStruct((M, D), jnp.float32),
            jax.ShapeDtypeStruct((S, D), jnp.float32),
        ],
        scratch_types=[
            pltpu.VMEM((_JOB_PER_W,), jnp.int32),
            pltpu.VMEM((_JOB_PER_W, D), jnp.float32),
            pltpu.VMEM((_SM_PER_W,), jnp.int32),
            pltpu.VMEM((_SM_PER_W, D), jnp.float32),
            pltpu.VMEM((_SM_PER_W,), jnp.int32),
            pltpu.VMEM((_SM_PER_W, D), jnp.float32),
            pltpu.SemaphoreType.DMA,
            pltpu.SemaphoreType.DMA,
            pltpu.SemaphoreType.DMA,
        ],
    )


def _sims_body(jemb_ref, memb_ref, semb_ref, jm_ref, ms_ref):
    je = jemb_ref[...]
    me = memb_ref[...]
    se = semb_ref[...]
    jn = jnp.sqrt(jnp.sum(je * je, axis=1))
    mn = jnp.sqrt(jnp.sum(me * me, axis=1))
    sn = jnp.sqrt(jnp.sum(se * se, axis=1))
    jm_dot = lax.dot_general(je, me, (((1,), (1,)), ((), ())),
                             preferred_element_type=jnp.float32)
    jm_ref[...] = jm_dot / jnp.maximum(jn[:, None] * mn[None, :], _EPS)
    ms_dot = lax.dot_general(me, se, (((1,), (1,)), ((), ())),
                             preferred_element_type=jnp.float32)
    ms_ref[...] = ms_dot / jnp.maximum(mn[:, None] * sn[None, :], _EPS)


def _outer_body(jm_hbm, ms_hbm, out_hbm, jm_v, ms_v, buf0_v, buf1_v,
                sem_in, sem_a, sem_b):
    wid = _wid()
    jbase = pl.multiple_of(wid * _JOB_PER_W, 8)
    i1 = pltpu.async_copy(jm_hbm.at[pl.ds(jbase, _JOB_PER_W)], jm_v, sem_in)
    i2 = pltpu.async_copy(ms_hbm, ms_v, sem_a)
    i1.wait()
    i2.wait()

    bufs = (buf0_v, buf1_v)
    sems = (sem_a, sem_b)
    pending = [None, None]

    def _fill(buf, c):
        for jj in range(_CHUNK_JOBS):
            jrel = c * _CHUNK_JOBS + jj

            def body(m, _):
                row16 = jm_v[jrel, pl.ds((m // _L) * _L, _L)]
                scale = lax.gather(
                    row16, jnp.full((_L, 1), m % _L, jnp.int32),
                    lax.GatherDimensionNumbers(
                        offset_dims=(), collapsed_slice_dims=(0,),
                        start_index_map=(0,)),
                    slice_sizes=(1,),
                    mode=lax.GatherScatterMode.PROMISE_IN_BOUNDS)
                for k in range(D // _L):
                    buf[jj, m, pl.ds(k * _L, _L)] = (
                        scale * ms_v[m, pl.ds(k * _L, _L)])
                return 0

            lax.fori_loop(0, M, body, 0)

    for c in range(_NCH):
        slot = c % 2
        if pending[slot] is not None:
            pending[slot].wait()
        _fill(bufs[slot], c)
        pending[slot] = pltpu.async_copy(
            bufs[slot],
            out_hbm.at[pl.ds(jbase + c * _CHUNK_JOBS, _CHUNK_JOBS)],
            sems[slot])
    pending[0].wait()
    pending[1].wait()


@functools.cache
def _outer_sc():
    return pl.kernel(
        _outer_body,
        mesh=plsc.VectorSubcoreMesh(core_axis_name="c", subcore_axis_name="s"),
        out_type=jax.ShapeDtypeStruct((J, M, S), jnp.float32),
        scratch_types=[
            pltpu.VMEM((_JOB_PER_W, M), jnp.float32),
            pltpu.VMEM((M, S), jnp.float32),
            pltpu.VMEM((_CHUNK_JOBS, M, S), jnp.float32),
            pltpu.VMEM((_CHUNK_JOBS, M, S), jnp.float32),
            pltpu.SemaphoreType.DMA,
            pltpu.SemaphoreType.DMA,
            pltpu.SemaphoreType.DMA,
        ],
    )


def kernel(job_indices, major_indices, subject_indices,
           job_table, major_table, subject_table):
    jemb, memb, semb = _gather_sc()(
        job_indices.astype(jnp.int32),
        major_indices.astype(jnp.int32).reshape(_NH, _SM_PER_W),
        subject_indices.astype(jnp.int32).reshape(_NH, _SM_PER_W),
        job_table, major_table, subject_table)
    jm, ms = pl.pallas_call(
        _sims_body,
        out_shape=[
            jax.ShapeDtypeStruct((J, M), jnp.float32),
            jax.ShapeDtypeStruct((M, S), jnp.float32),
        ],
    )(jemb, memb, semb)
    out = _outer_sc()(jm, ms)
    return out.reshape(-1)


# speedup vs baseline: 1.5040x; 1.5040x over previous
import functools

import jax
import jax.numpy as jnp
from jax import lax
from jax.experimental import pallas as pl
from jax.experimental.pallas import tpu as pltpu
from jax.experimental.pallas import tpu_sc as plsc

J, M, S, D = 1024, 128, 128, 128
_EPS = 1e-8

_NC, _NS = 2, 16
_NW = _NC * _NS
_NH = _NW // 2
_SM_PER_W = M // _NH
_JOB_PER_W = J // _NW
_L = 16
_CHUNK_JOBS = 2
_NCH = _JOB_PER_W // _CHUNK_JOBS


def _wid():
    return lax.axis_index("s") * _NC + lax.axis_index("c")


def _gather_body(jidx_hbm, midx_hbm, sidx_hbm, jtab_hbm, mtab_hbm, stab_hbm,
                 jout_hbm, mout_hbm, sout_hbm,
                 jidx_v, jrows_v, midx_v, mrows_v, sidx_v, srows_v,
                 sem_a, sem_b, sem_c):
    wid = _wid()
    hid = lax.rem(wid, _NH)
    jb = pl.multiple_of(wid * _JOB_PER_W, 8)
    sb = pl.multiple_of(hid * _SM_PER_W, 8)

    c1 = pltpu.async_copy(jidx_hbm.at[pl.ds(jb, _JOB_PER_W)], jidx_v, sem_a)
    c2 = pltpu.async_copy(midx_hbm.at[hid], midx_v, sem_b)
    c3 = pltpu.async_copy(sidx_hbm.at[hid], sidx_v, sem_c)
    c1.wait()
    c2.wait()
    c3.wait()
    g1 = pltpu.async_copy(jtab_hbm.at[jidx_v], jrows_v, sem_a)
    g2 = pltpu.async_copy(mtab_hbm.at[midx_v], mrows_v, sem_b)
    g3 = pltpu.async_copy(stab_hbm.at[sidx_v], srows_v, sem_c)
    g1.wait()
    g2.wait()
    g3.wait()
    w1 = pltpu.async_copy(jrows_v, jout_hbm.at[pl.ds(jb, _JOB_PER_W)], sem_a)
    w2 = pltpu.async_copy(mrows_v, mout_hbm.at[pl.ds(sb, _SM_PER_W)], sem_b)
    w3 = pltpu.async_copy(srows_v, sout_hbm.at[pl.ds(sb, _SM_PER_W)], sem_c)
    w1.wait()
    w2.wait()
    w3.wait()


@functools.cache
def _gather_sc():
    return pl.kernel(
        _gather_body,
        mesh=plsc.VectorSubcoreMesh(core_axis_name="c", subcore_axis_name="s"),
        out_type=[
            jax.ShapeDtypeStruct((J, D), jnp.float32),
            jax.ShapeDtypeStruct((M, D), jnp.float32),
            jax.ShapeDtypeStruct((S, D), jnp.float32),
        ],
        scratch_types=[
            pltpu.VMEM((_JOB_PER_W,), jnp.int32),
            pltpu.VMEM((_JOB_PER_W, D), jnp.float32),
            pltpu.VMEM((_SM_PER_W,), jnp.int32),
            pltpu.VMEM((_SM_PER_W, D), jnp.float32),
            pltpu.VMEM((_SM_PER_W,), jnp.int32),
            pltpu.VMEM((_SM_PER_W, D), jnp.float32),
            pltpu.SemaphoreType.DMA,
            pltpu.SemaphoreType.DMA,
            pltpu.SemaphoreType.DMA,
        ],
    )


def _sims_body(jemb_ref, memb_ref, semb_ref, jm_ref, ms_ref):
    je = jemb_ref[...]
    me = memb_ref[...]
    se = semb_ref[...]
    jn = jnp.sqrt(jnp.sum(je * je, axis=1))
    mn = jnp.sqrt(jnp.sum(me * me, axis=1))
    sn = jnp.sqrt(jnp.sum(se * se, axis=1))
    jm_dot = lax.dot_general(je, me, (((1,), (1,)), ((), ())),
                             preferred_element_type=jnp.float32)
    jm_ref[...] = jm_dot / jnp.maximum(jn[:, None] * mn[None, :], _EPS)
    ms_dot = lax.dot_general(me, se, (((1,), (1,)), ((), ())),
                             preferred_element_type=jnp.float32)
    ms_ref[...] = ms_dot / jnp.maximum(mn[:, None] * sn[None, :], _EPS)


def _splat(row16, lane):
    return lax.gather(
        row16, jnp.full((_L, 1), lane, jnp.int32),
        lax.GatherDimensionNumbers(
            offset_dims=(), collapsed_slice_dims=(0,), start_index_map=(0,)),
        slice_sizes=(1,),
        mode=lax.GatherScatterMode.PROMISE_IN_BOUNDS)


def _outer_body(jm_hbm, ms_hbm, out_hbm, jm_v, ms_v, buf0_v, buf1_v,
                sem_in, sem_a, sem_b):
    wid = _wid()
    jbase = pl.multiple_of(wid * _JOB_PER_W, 8)
    i1 = pltpu.async_copy(jm_hbm.at[pl.ds(jbase, _JOB_PER_W)], jm_v, sem_in)
    i2 = pltpu.async_copy(ms_hbm, ms_v, sem_in)
    i1.wait()
    i2.wait()

    def _fill(buf, c):
        for jj in range(_CHUNK_JOBS):
            jrel = c * _CHUNK_JOBS + jj

            def gbody(g, _):
                row16 = jm_v[jrel, pl.ds(g * _L, _L)]
                for i in range(_L):
                    scale = _splat(row16, i)
                    mrow = g * _L + i
                    for k in range(D // _L):
                        buf[jj, mrow, pl.ds(k * _L, _L)] = (
                            scale * ms_v[mrow, pl.ds(k * _L, _L)])
                return 0

            lax.fori_loop(0, M // _L, gbody, 0)

    def _fire(buf, c, sem):
        pltpu.async_copy(
            buf, out_hbm.at[pl.ds(jbase + c * _CHUNK_JOBS, _CHUNK_JOBS)], sem)

    def _drain(buf, sem):
        pltpu.make_async_copy(out_hbm.at[pl.ds(jbase, _CHUNK_JOBS)], buf,
                              sem).wait()

    _fill(buf0_v, 0)
    _fire(buf0_v, 0, sem_a)
    _fill(buf1_v, 1)
    _fire(buf1_v, 1, sem_b)

    def pbody(p, _):
        c0 = p * 2
        _drain(buf0_v, sem_a)
        _fill(buf0_v, c0)
        _fire(buf0_v, c0, sem_a)
        _drain(buf1_v, sem_b)
        _fill(buf1_v, c0 + 1)
        _fire(buf1_v, c0 + 1, sem_b)
        return 0

    lax.fori_loop(1, _NCH // 2, pbody, 0)
    _drain(buf0_v, sem_a)
    _drain(buf1_v, sem_b)


@functools.cache
def _outer_sc():
    return pl.kernel(
        _outer_body,
        mesh=plsc.VectorSubcoreMesh(core_axis_name="c", subcore_axis_name="s"),
        out_type=jax.ShapeDtypeStruct((J, M, S), jnp.float32),
        scratch_types=[
            pltpu.VMEM((_JOB_PER_W, M), jnp.float32),
            pltpu.VMEM((M, S), jnp.float32),
            pltpu.VMEM((_CHUNK_JOBS, M, S), jnp.float32),
            pltpu.VMEM((_CHUNK_JOBS, M, S), jnp.float32),
            pltpu.SemaphoreType.DMA,
            pltpu.SemaphoreType.DMA,
            pltpu.SemaphoreType.DMA,
        ],
    )


def kernel(job_indices, major_indices, subject_indices,
           job_table, major_table, subject_table):
    jemb, memb, semb = _gather_sc()(
        job_indices.astype(jnp.int32),
        major_indices.astype(jnp.int32).reshape(_NH, _SM_PER_W),
        subject_indices.astype(jnp.int32).reshape(_NH, _SM_PER_W),
        job_table, major_table, subject_table)
    jm, ms = pl.pallas_call(
        _sims_body,
        out_shape=[
            jax.ShapeDtypeStruct((J, M), jnp.float32),
            jax.ShapeDtypeStruct((M, S), jnp.float32),
        ],
    )(jemb, memb, semb)
    out = _outer_sc()(jm, ms)
    return out.reshape(-1)


# speedup vs baseline: 2.6745x; 1.7782x over previous
import functools

import jax
import jax.numpy as jnp
from jax import lax
from jax.experimental import pallas as pl
from jax.experimental.pallas import tpu as pltpu
from jax.experimental.pallas import tpu_sc as plsc

J, M, S, D = 1024, 128, 128, 128
_EPS = 1e-8

_NC, _NS = 2, 16
_NW = _NC * _NS
_JOB_PER_W = J // _NW
_NH = _NW // 2
_SM_PER_W = M // _NH


def _gather_body(jidx_hbm, midx_hbm, sidx_hbm, jtab_hbm, mtab_hbm, stab_hbm,
                 jout_hbm, mout_hbm, sout_hbm,
                 jidx_v, jrows_v, midx_v, mrows_v, sidx_v, srows_v,
                 sem_a, sem_b, sem_c):
    wid = lax.axis_index("s") * _NC + lax.axis_index("c")
    hid = lax.rem(wid, _NH)

    jb = pl.multiple_of(wid * _JOB_PER_W, 8)
    sb = pl.multiple_of(hid * _SM_PER_W, 8)

    c1 = pltpu.async_copy(jidx_hbm.at[pl.ds(jb, _JOB_PER_W)], jidx_v, sem_a)
    c2 = pltpu.async_copy(midx_hbm.at[hid], midx_v, sem_b)
    c3 = pltpu.async_copy(sidx_hbm.at[hid], sidx_v, sem_c)
    c1.wait()
    c2.wait()
    c3.wait()
    g1 = pltpu.async_copy(jtab_hbm.at[jidx_v], jrows_v, sem_a)
    g2 = pltpu.async_copy(mtab_hbm.at[midx_v], mrows_v, sem_b)
    g3 = pltpu.async_copy(stab_hbm.at[sidx_v], srows_v, sem_c)
    g1.wait()
    g2.wait()
    g3.wait()
    w1 = pltpu.async_copy(jrows_v, jout_hbm.at[pl.ds(jb, _JOB_PER_W)], sem_a)
    w2 = pltpu.async_copy(mrows_v, mout_hbm.at[pl.ds(sb, _SM_PER_W)], sem_b)
    w3 = pltpu.async_copy(srows_v, sout_hbm.at[pl.ds(sb, _SM_PER_W)], sem_c)
    w1.wait()
    w2.wait()
    w3.wait()


@functools.cache
def _gather_sc():
    return pl.kernel(
        _gather_body,
        mesh=plsc.VectorSubcoreMesh(core_axis_name="c", subcore_axis_name="s"),
        out_type=[
            jax.ShapeDtypeStruct((J, D), jnp.float32),
            jax.ShapeDtypeStruct((M, D), jnp.float32),
            jax.ShapeDtypeStruct((S, D), jnp.float32),
        ],
        scratch_types=[
            pltpu.VMEM((_JOB_PER_W,), jnp.int32),
            pltpu.VMEM((_JOB_PER_W, D), jnp.float32),
            pltpu.VMEM((_SM_PER_W,), jnp.int32),
            pltpu.VMEM((_SM_PER_W, D), jnp.float32),
            pltpu.VMEM((_SM_PER_W,), jnp.int32),
            pltpu.VMEM((_SM_PER_W, D), jnp.float32),
            pltpu.SemaphoreType.DMA,
            pltpu.SemaphoreType.DMA,
            pltpu.SemaphoreType.DMA,
        ],
    )


def _sim_body(jemb_ref, memb_ref, semb_ref, out_ref):
    je = jemb_ref[...]
    me = memb_ref[...]
    se = semb_ref[...]
    jn = jnp.sqrt(jnp.sum(je * je, axis=1))
    mn = jnp.sqrt(jnp.sum(me * me, axis=1))
    sn = jnp.sqrt(jnp.sum(se * se, axis=1))
    jm_dot = lax.dot_general(je, me, (((1,), (1,)), ((), ())),
                             preferred_element_type=jnp.float32)
    jm = jm_dot / jnp.maximum(jn[:, None] * mn[None, :], _EPS)
    ms_dot = lax.dot_general(me, se, (((1,), (1,)), ((), ())),
                             preferred_element_type=jnp.float32)
    ms = ms_dot / jnp.maximum(mn[:, None] * sn[None, :], _EPS)
    out_ref[...] = jm[:, :, None] * ms[None, :, :]


_JB = 64


def kernel(job_indices, major_indices, subject_indices,
           job_table, major_table, subject_table):
    jemb, memb, semb = _gather_sc()(
        job_indices.astype(jnp.int32),
        major_indices.astype(jnp.int32).reshape(_NH, _SM_PER_W),
        subject_indices.astype(jnp.int32).reshape(_NH, _SM_PER_W),
        job_table, major_table, subject_table)
    out = pl.pallas_call(
        _sim_body,
        grid=(J // _JB,),
        in_specs=[
            pl.BlockSpec((_JB, D), lambda i: (i, 0)),
            pl.BlockSpec((M, D), lambda i: (0, 0)),
            pl.BlockSpec((S, D), lambda i: (0, 0)),
        ],
        out_specs=pl.BlockSpec((_JB, M, S), lambda i: (i, 0, 0)),
        out_shape=jax.ShapeDtypeStruct((J, M, S), jnp.float32),
    )(jemb, memb, semb)
    return out.reshape(-1)


# speedup vs baseline: 2.7628x; 1.0330x over previous
import functools

import jax
import jax.numpy as jnp
from jax import lax
from jax.experimental import pallas as pl
from jax.experimental.pallas import tpu as pltpu
from jax.experimental.pallas import tpu_sc as plsc

J, M, S, D = 1024, 128, 128, 128
_EPS = 1e-8

_NC, _NS = 2, 16
_NW = _NC * _NS
_JOB_PER_W = J // _NW
_NH = _NW // 2
_SM_PER_W = M // _NH


def _gather_body(jidx_hbm, midx_hbm, sidx_hbm, jtab_hbm, mtab_hbm, stab_hbm,
                 jout_hbm, mout_hbm, sout_hbm,
                 jidx_v, jrows_v, midx_v, mrows_v, sidx_v, srows_v,
                 sem_a, sem_b, sem_c):
    wid = lax.axis_index("s") * _NC + lax.axis_index("c")
    hid = lax.rem(wid, _NH)

    jb = pl.multiple_of(wid * _JOB_PER_W, 8)
    sb = pl.multiple_of(hid * _SM_PER_W, 8)

    c1 = pltpu.async_copy(jidx_hbm.at[pl.ds(jb, _JOB_PER_W)], jidx_v, sem_a)
    c2 = pltpu.async_copy(midx_hbm.at[hid], midx_v, sem_b)
    c3 = pltpu.async_copy(sidx_hbm.at[hid], sidx_v, sem_c)
    c1.wait()
    c2.wait()
    c3.wait()
    g1 = pltpu.async_copy(jtab_hbm.at[jidx_v], jrows_v, sem_a)
    g2 = pltpu.async_copy(mtab_hbm.at[midx_v], mrows_v, sem_b)
    g3 = pltpu.async_copy(stab_hbm.at[sidx_v], srows_v, sem_c)
    g1.wait()
    g2.wait()
    g3.wait()
    w1 = pltpu.async_copy(jrows_v, jout_hbm.at[pl.ds(jb, _JOB_PER_W)], sem_a)
    w2 = pltpu.async_copy(mrows_v, mout_hbm.at[pl.ds(sb, _SM_PER_W)], sem_b)
    w3 = pltpu.async_copy(srows_v, sout_hbm.at[pl.ds(sb, _SM_PER_W)], sem_c)
    w1.wait()
    w2.wait()
    w3.wait()


@functools.cache
def _gather_sc():
    return pl.kernel(
        _gather_body,
        mesh=plsc.VectorSubcoreMesh(core_axis_name="c", subcore_axis_name="s"),
        out_type=[
            jax.ShapeDtypeStruct((J, D), jnp.float32),
            jax.ShapeDtypeStruct((M, D), jnp.float32),
            jax.ShapeDtypeStruct((S, D), jnp.float32),
        ],
        scratch_types=[
            pltpu.VMEM((_JOB_PER_W,), jnp.int32),
            pltpu.VMEM((_JOB_PER_W, D), jnp.float32),
            pltpu.VMEM((_SM_PER_W,), jnp.int32),
            pltpu.VMEM((_SM_PER_W, D), jnp.float32),
            pltpu.VMEM((_SM_PER_W,), jnp.int32),
            pltpu.VMEM((_SM_PER_W, D), jnp.float32),
            pltpu.SemaphoreType.DMA,
            pltpu.SemaphoreType.DMA,
            pltpu.SemaphoreType.DMA,
        ],
    )


def _sim_body(jemb_ref, memb_ref, semb_ref, out_ref):
    je = jemb_ref[...]
    me = memb_ref[...]
    se = semb_ref[...]
    jn = jnp.sqrt(jnp.sum(je * je, axis=1))
    mn = jnp.sqrt(jnp.sum(me * me, axis=1))
    sn = jnp.sqrt(jnp.sum(se * se, axis=1))
    jm_dot = lax.dot_general(je, me, (((1,), (1,)), ((), ())),
                             preferred_element_type=jnp.float32)
    jm = jm_dot / jnp.maximum(jn[:, None] * mn[None, :], _EPS)
    ms_dot = lax.dot_general(me, se, (((1,), (1,)), ((), ())),
                             preferred_element_type=jnp.float32)
    ms = ms_dot / jnp.maximum(mn[:, None] * sn[None, :], _EPS)
    out_ref[...] = jm[:, :, None] * ms[None, :, :]


_JB = 256


def kernel(job_indices, major_indices, subject_indices,
           job_table, major_table, subject_table):
    jemb, memb, semb = _gather_sc()(
        job_indices.astype(jnp.int32),
        major_indices.astype(jnp.int32).reshape(_NH, _SM_PER_W),
        subject_indices.astype(jnp.int32).reshape(_NH, _SM_PER_W),
        job_table, major_table, subject_table)
    out = pl.pallas_call(
        _sim_body,
        grid=(J // _JB,),
        in_specs=[
            pl.BlockSpec((_JB, D), lambda i: (i, 0)),
            pl.BlockSpec((M, D), lambda i: (0, 0)),
            pl.BlockSpec((S, D), lambda i: (0, 0)),
        ],
        out_specs=pl.BlockSpec((_JB, M, S), lambda i: (i, 0, 0)),
        out_shape=jax.ShapeDtypeStruct((J, M, S), jnp.float32),
    )(jemb, memb, semb)
    return out.reshape(-1)


# speedup vs baseline: 2.8230x; 1.0218x over previous
import functools

import jax
import jax.numpy as jnp
from jax import lax
from jax.experimental import pallas as pl
from jax.experimental.pallas import tpu as pltpu
from jax.experimental.pallas import tpu_sc as plsc

J, M, S, D = 1024, 128, 128, 128
_EPS = 1e-8

_NC, _NS = 2, 16
_NW = _NC * _NS
_JOB_PER_W = J // _NW
_NH = _NW // 2
_SM_PER_W = M // _NH
_IDX_PER_W = _JOB_PER_W + 2 * _SM_PER_W
_MBASE = J
_SBASE = J + M


def _gather_body(widx_hbm, jtab_hbm, mtab_hbm, stab_hbm, emb_hbm,
                 idx_v, jrows_v, mrows_v, srows_v, sem_a, sem_b, sem_c):
    wid = lax.axis_index("s") * _NC + lax.axis_index("c")
    hid = lax.rem(wid, _NH)
    jb = pl.multiple_of(wid * _JOB_PER_W, 8)
    mb = pl.multiple_of(_MBASE + hid * _SM_PER_W, 8)
    sb = pl.multiple_of(_SBASE + hid * _SM_PER_W, 8)

    pltpu.async_copy(widx_hbm.at[wid], idx_v, sem_a).wait()
    g1 = pltpu.async_copy(jtab_hbm.at[idx_v.at[pl.ds(0, _JOB_PER_W)]],
                          jrows_v, sem_a)
    g2 = pltpu.async_copy(
        mtab_hbm.at[idx_v.at[pl.ds(_JOB_PER_W, _SM_PER_W)]], mrows_v, sem_b)
    g3 = pltpu.async_copy(
        stab_hbm.at[idx_v.at[pl.ds(_JOB_PER_W + _SM_PER_W, _SM_PER_W)]],
        srows_v, sem_c)
    g1.wait()
    g2.wait()
    g3.wait()
    w1 = pltpu.async_copy(jrows_v, emb_hbm.at[pl.ds(jb, _JOB_PER_W)], sem_a)
    w2 = pltpu.async_copy(mrows_v, emb_hbm.at[pl.ds(mb, _SM_PER_W)], sem_b)
    w3 = pltpu.async_copy(srows_v, emb_hbm.at[pl.ds(sb, _SM_PER_W)], sem_c)
    w1.wait()
    w2.wait()
    w3.wait()


@functools.cache
def _gather_sc():
    return pl.kernel(
        _gather_body,
        mesh=plsc.VectorSubcoreMesh(core_axis_name="c", subcore_axis_name="s"),
        out_type=jax.ShapeDtypeStruct((J + M + S, D), jnp.float32),
        scratch_types=[
            pltpu.VMEM((_IDX_PER_W,), jnp.int32),
            pltpu.VMEM((_JOB_PER_W, D), jnp.float32),
            pltpu.VMEM((_SM_PER_W, D), jnp.float32),
            pltpu.VMEM((_SM_PER_W, D), jnp.float32),
            pltpu.SemaphoreType.DMA,
            pltpu.SemaphoreType.DMA,
            pltpu.SemaphoreType.DMA,
        ],
    )


def _sim_body(jemb_ref, memb_ref, semb_ref, out_ref):
    je = jemb_ref[...]
    me = memb_ref[...]
    se = semb_ref[...]
    jn = jnp.sqrt(jnp.sum(je * je, axis=1))
    mn = jnp.sqrt(jnp.sum(me * me, axis=1))
    sn = jnp.sqrt(jnp.sum(se * se, axis=1))
    jm_dot = lax.dot_general(je, me, (((1,), (1,)), ((), ())),
                             preferred_element_type=jnp.float32)
    jm = jm_dot / jnp.maximum(jn[:, None] * mn[None, :], _EPS)
    ms_dot = lax.dot_general(me, se, (((1,), (1,)), ((), ())),
                             preferred_element_type=jnp.float32)
    ms = ms_dot / jnp.maximum(mn[:, None] * sn[None, :], _EPS)
    out_ref[...] = jm[:, :, None] * ms[None, :, :]


_JB = 128


def kernel(job_indices, major_indices, subject_indices,
           job_table, major_table, subject_table):
    jidx = job_indices.astype(jnp.int32).reshape(_NW, _JOB_PER_W)
    midx = jnp.tile(major_indices.astype(jnp.int32).reshape(_NH, _SM_PER_W),
                    (2, 1))
    sidx = jnp.tile(subject_indices.astype(jnp.int32).reshape(_NH, _SM_PER_W),
                    (2, 1))
    widx = jnp.concatenate([jidx, midx, sidx], axis=1)
    emb = _gather_sc()(widx, job_table, major_table, subject_table)
    out = pl.pallas_call(
        _sim_body,
        grid=(J // _JB,),
        in_specs=[
            pl.BlockSpec((_JB, D), lambda i: (i, 0)),
            pl.BlockSpec((M, D), lambda i: (_MBASE // M, 0)),
            pl.BlockSpec((S, D), lambda i: (_SBASE // S, 0)),
        ],
        out_specs=pl.BlockSpec((_JB, M, S), lambda i: (i, 0, 0)),
        out_shape=jax.ShapeDtypeStruct((J, M, S), jnp.float32),
    )(emb, emb, emb)
    return out.reshape(-1)


# speedup vs baseline: 2.8394x; 1.0058x over previous
import functools

import jax
import jax.numpy as jnp
from jax import lax
from jax.experimental import pallas as pl
from jax.experimental.pallas import tpu as pltpu
from jax.experimental.pallas import tpu_sc as plsc

J, M, S, D = 1024, 128, 128, 128
_EPS = 1e-8

_NC, _NS = 2, 16
_NW = _NC * _NS
_JOB_PER_W = J // _NW
_NH = _NW // 2
_SM_PER_W = M // _NH


def _gather_body(jidx_hbm, midx_hbm, sidx_hbm, jtab_hbm, mtab_hbm, stab_hbm,
                 jout_hbm, mout_hbm, sout_hbm,
                 jidx_v, jrows_v, midx_v, mrows_v, sidx_v, srows_v,
                 sem_a, sem_b, sem_c):
    wid = lax.axis_index("s") * _NC + lax.axis_index("c")
    hid = lax.rem(wid, _NH)

    jb = pl.multiple_of(wid * _JOB_PER_W, 8)
    sb = pl.multiple_of(hid * _SM_PER_W, 8)

    c1 = pltpu.async_copy(jidx_hbm.at[pl.ds(jb, _JOB_PER_W)], jidx_v, sem_a)
    c2 = pltpu.async_copy(midx_hbm.at[hid], midx_v, sem_b)
    c3 = pltpu.async_copy(sidx_hbm.at[hid], sidx_v, sem_c)
    c2.wait()
    g2 = pltpu.async_copy(mtab_hbm.at[midx_v], mrows_v, sem_b)
    c3.wait()
    g3 = pltpu.async_copy(stab_hbm.at[sidx_v], srows_v, sem_c)
    c1.wait()
    g1 = pltpu.async_copy(jtab_hbm.at[jidx_v], jrows_v, sem_a)
    g2.wait()
    w2 = pltpu.async_copy(mrows_v, mout_hbm.at[pl.ds(sb, _SM_PER_W)], sem_b)
    g3.wait()
    w3 = pltpu.async_copy(srows_v, sout_hbm.at[pl.ds(sb, _SM_PER_W)], sem_c)
    g1.wait()
    w1 = pltpu.async_copy(jrows_v, jout_hbm.at[pl.ds(jb, _JOB_PER_W)], sem_a)
    w2.wait()
    w3.wait()
    w1.wait()


@functools.cache
def _gather_sc():
    return pl.kernel(
        _gather_body,
        mesh=plsc.VectorSubcoreMesh(core_axis_name="c", subcore_axis_name="s"),
        out_type=[
            jax.ShapeDtypeStruct((J, D), jnp.float32),
            jax.ShapeDtypeStruct((M, D), jnp.float32),
            jax.ShapeDtypeStruct((S, D), jnp.float32),
        ],
        scratch_types=[
            pltpu.VMEM((_JOB_PER_W,), jnp.int32),
            pltpu.VMEM((_JOB_PER_W, D), jnp.float32),
            pltpu.VMEM((_SM_PER_W,), jnp.int32),
            pltpu.VMEM((_SM_PER_W, D), jnp.float32),
            pltpu.VMEM((_SM_PER_W,), jnp.int32),
            pltpu.VMEM((_SM_PER_W, D), jnp.float32),
            pltpu.SemaphoreType.DMA,
            pltpu.SemaphoreType.DMA,
            pltpu.SemaphoreType.DMA,
        ],
    )


def _sim_body(jemb_ref, memb_ref, semb_ref, out_ref):
    je = jemb_ref[...]
    me = memb_ref[...]
    se = semb_ref[...]
    jn = jnp.sqrt(jnp.sum(je * je, axis=1))
    mn = jnp.sqrt(jnp.sum(me * me, axis=1))
    sn = jnp.sqrt(jnp.sum(se * se, axis=1))
    jm_dot = lax.dot_general(je, me, (((1,), (1,)), ((), ())),
                             preferred_element_type=jnp.float32)
    jm = jm_dot / jnp.maximum(jn[:, None] * mn[None, :], _EPS)
    ms_dot = lax.dot_general(me, se, (((1,), (1,)), ((), ())),
                             preferred_element_type=jnp.float32)
    ms = ms_dot / jnp.maximum(mn[:, None] * sn[None, :], _EPS)
    out_ref[...] = jm[:, :, None] * ms[None, :, :]


_JB = 128


def kernel(job_indices, major_indices, subject_indices,
           job_table, major_table, subject_table):
    jemb, memb, semb = _gather_sc()(
        job_indices.astype(jnp.int32),
        major_indices.astype(jnp.int32).reshape(_NH, _SM_PER_W),
        subject_indices.astype(jnp.int32).reshape(_NH, _SM_PER_W),
        job_table, major_table, subject_table)
    out = pl.pallas_call(
        _sim_body,
        grid=(J // _JB,),
        in_specs=[
            pl.BlockSpec((_JB, D), lambda i: (i, 0)),
            pl.BlockSpec((M, D), lambda i: (0, 0)),
            pl.BlockSpec((S, D), lambda i: (0, 0)),
        ],
        out_specs=pl.BlockSpec((_JB, M, S), lambda i: (i, 0, 0)),
        out_shape=jax.ShapeDtypeStruct((J, M, S), jnp.float32),
    )(jemb, memb, semb)
    return out.reshape(-1)
